# Initial kernel scaffold; baseline (speedup 1.0000x reference)
import jax
import jax.numpy as jnp
from jax.experimental import pallas as pl


def kernel(node_feats, node_attrs, edge_feats, edge_index, W_conv, W_lin, W_skip):
    raise NotImplementedError("write your pallas kernel here")


# speedup vs baseline: 1.4156x; 1.4156x over previous
import functools

import jax
import jax.numpy as jnp
from jax import lax
from jax.experimental import pallas as pl
from jax.experimental.pallas import tpu as pltpu
from jax.experimental.pallas import tpu_sc as plsc

N = 10000
E = 320000
D_NODE = 128
D_ATTR = 16
D_EDGE = 4
D_MID = 128
D_OUT = 128

NC = 2
NS = 16
NW = NC * NS
C = 64
NCHUNK = E // C
NPAD = 10240
ROWS_PER_TILE = NPAD // NS

_INV_CONV = 1.0 / (D_NODE * D_EDGE) ** 0.5
_INV_LIN = 1.0 / D_MID**0.5
_INV_SKIP = 1.0 / (D_OUT * D_ATTR) ** 0.5


def _conv_proj_body(nf_ref, w_ref, o_ref):
    o_ref[...] = (
        jnp.dot(nf_ref[...], w_ref[...], preferred_element_type=jnp.float32)
        * _INV_CONV
    )


def _conv_proj(node_feats, w2):
    bn = 400
    return pl.pallas_call(
        _conv_proj_body,
        grid=(N // bn,),
        in_specs=[
            pl.BlockSpec((bn, D_NODE), lambda i: (i, 0)),
            pl.BlockSpec((D_NODE, D_EDGE * D_MID), lambda i: (0, 0)),
        ],
        out_specs=pl.BlockSpec((bn, D_EDGE * D_MID), lambda i: (i, 0)),
        out_shape=jax.ShapeDtypeStruct((N, D_EDGE * D_MID), jnp.float32),
    )(node_feats, w2)


def _sc_body(p_hbm, send_hbm, recv_hbm, ef_hbm, zero_hbm, out_hbm,
             idx_s, idx_r, ef_b, prow, outb, acc, sem):
    cid = lax.axis_index("c")
    sid = lax.axis_index("s")
    wid = sid * NC + cid

    pltpu.sync_copy(
        zero_hbm.at[pl.ds(sid * ROWS_PER_TILE, ROWS_PER_TILE)],
        acc.at[pl.ds(sid * ROWS_PER_TILE, ROWS_PER_TILE)],
    )
    plsc.subcore_barrier()

    n_i = (NCHUNK - wid + NW - 1) // NW

    def chunk_body(i, _):
        base = (wid + i * NW) * C
        pltpu.sync_copy(send_hbm.at[pl.ds(base, C)], idx_s)
        pltpu.sync_copy(recv_hbm.at[pl.ds(base, C)], idx_r)
        pltpu.sync_copy(ef_hbm.at[pl.ds(base * D_EDGE, C * D_EDGE)], ef_b)
        pltpu.async_copy(p_hbm.at[idx_s], prow, sem).wait()

        def group_body(q, _):
            v = ef_b[pl.ds(16 * q, 16)]
            for r in range(4):
                e = 4 * q + r
                f0 = v[4 * r + 0]
                f1 = v[4 * r + 1]
                f2 = v[4 * r + 2]
                f3 = v[4 * r + 3]
                for kg in range(8):
                    o = kg * 16
                    w = f0 * prow[e, pl.ds(o, 16)]
                    w = w + f1 * prow[e, pl.ds(128 + o, 16)]
                    w = w + f2 * prow[e, pl.ds(256 + o, 16)]
                    w = w + f3 * prow[e, pl.ds(384 + o, 16)]
                    outb[e, pl.ds(o, 16)] = w
            return 0

        lax.fori_loop(0, C // 4, group_body, 0)
        pltpu.sync_copy(outb, acc.at[idx_r], add=True)
        return 0

    lax.fori_loop(0, n_i, chunk_body, 0)

    plsc.subcore_barrier()
    pltpu.sync_copy(
        acc.at[pl.ds(sid * ROWS_PER_TILE, ROWS_PER_TILE)],
        out_hbm.at[cid, pl.ds(sid * ROWS_PER_TILE, ROWS_PER_TILE)],
    )


@functools.lru_cache(maxsize=1)
def _sc_scatter():
    return pl.kernel(
        _sc_body,
        out_type=jax.ShapeDtypeStruct((NC, NPAD, D_MID), jnp.float32),
        mesh=plsc.VectorSubcoreMesh(
            core_axis_name="c", subcore_axis_name="s",
            num_cores=NC, num_subcores=NS,
        ),
        scratch_types=[
            pltpu.VMEM((C,), jnp.int32),
            pltpu.VMEM((C,), jnp.int32),
            pltpu.VMEM((C * D_EDGE,), jnp.float32),
            pltpu.VMEM((C, D_EDGE * D_MID), jnp.float32),
            pltpu.VMEM((C, D_MID), jnp.float32),
            pltpu.VMEM_SHARED((NPAD, D_MID), jnp.float32),
            pltpu.SemaphoreType.DMA,
        ],
    )


def _final_body(p0_ref, p1_ref, attr_ref, wl_ref, wsk_ref, o_ref):
    x = (
        jnp.dot(
            p0_ref[...] + p1_ref[...],
            wl_ref[...],
            preferred_element_type=jnp.float32,
        )
        * _INV_LIN
    )
    sk = jnp.zeros_like(x)
    for j in range(D_ATTR):
        sk = sk + jnp.dot(
            x * attr_ref[:, j : j + 1],
            wsk_ref[j],
            preferred_element_type=jnp.float32,
        )
    o_ref[...] = x + sk * _INV_SKIP


def _finalize(p0, p1, node_attrs, w_lin, wsk):
    bn = 400
    return pl.pallas_call(
        _final_body,
        grid=(N // bn,),
        in_specs=[
            pl.BlockSpec((bn, D_MID), lambda i: (i, 0)),
            pl.BlockSpec((bn, D_MID), lambda i: (i, 0)),
            pl.BlockSpec((bn, D_ATTR), lambda i: (i, 0)),
            pl.BlockSpec((D_MID, D_OUT), lambda i: (0, 0)),
            pl.BlockSpec((D_ATTR, D_MID, D_OUT), lambda i: (0, 0, 0)),
        ],
        out_specs=pl.BlockSpec((bn, D_OUT), lambda i: (i, 0)),
        out_shape=jax.ShapeDtypeStruct((N, D_OUT), jnp.float32),
    )(p0, p1, node_attrs, w_lin, wsk)


def kernel(node_feats, node_attrs, edge_feats, edge_index, W_conv, W_lin, W_skip):
    p = _conv_proj(node_feats, W_conv.reshape(D_NODE, D_EDGE * D_MID))
    zeros = jnp.zeros((NPAD, D_MID), jnp.float32)
    partial = _sc_scatter()(
        p, edge_index[0], edge_index[1], edge_feats.reshape(-1), zeros
    )
    return _finalize(
        partial[0], partial[1], node_attrs, W_lin, W_skip.transpose(1, 0, 2)
    )


# speedup vs baseline: 1.4769x; 1.0434x over previous
import functools

import jax
import jax.numpy as jnp
from jax import lax
from jax.experimental import pallas as pl
from jax.experimental.pallas import tpu as pltpu
from jax.experimental.pallas import tpu_sc as plsc

N = 10000
E = 320000
D_NODE = 128
D_ATTR = 16
D_EDGE = 4
D_MID = 128
D_OUT = 128

NC = 2
NS = 16
NW = NC * NS
C = 32
EPAD = 321536
NCHUNK = EPAD // C
NPC = NCHUNK // NW
NPAD = 10240
ROWS_PER_TILE = NPAD // NS

_INV_CONV = 1.0 / (D_NODE * D_EDGE) ** 0.5
_INV_LIN = 1.0 / D_MID**0.5
_INV_SKIP = 1.0 / (D_OUT * D_ATTR) ** 0.5


def _conv_proj_body(nf_ref, w_ref, o_ref):
    o_ref[...] = (
        jnp.dot(nf_ref[...], w_ref[...], preferred_element_type=jnp.float32)
        * _INV_CONV
    )


def _conv_proj(node_feats, w2):
    bn = 400
    return pl.pallas_call(
        _conv_proj_body,
        grid=(N // bn,),
        in_specs=[
            pl.BlockSpec((bn, D_NODE), lambda i: (i, 0)),
            pl.BlockSpec((D_NODE, D_EDGE * D_MID), lambda i: (0, 0)),
        ],
        out_specs=pl.BlockSpec((bn, D_EDGE * D_MID), lambda i: (i, 0)),
        out_shape=jax.ShapeDtypeStruct((N, D_EDGE * D_MID), jnp.float32),
    )(node_feats, w2)


def _sc_body(p_hbm, send_hbm, recv_hbm, ef_hbm, zero_hbm, out_hbm,
             idx_s0, idx_s1, idx_r0, idx_r1, ef_b0, ef_b1,
             prow0, prow1, outb, acc, sem0, sem1):
    cid = lax.axis_index("c")
    sid = lax.axis_index("s")
    wid = sid * NC + cid

    pltpu.sync_copy(
        zero_hbm.at[pl.ds(sid * ROWS_PER_TILE, ROWS_PER_TILE)],
        acc.at[pl.ds(sid * ROWS_PER_TILE, ROWS_PER_TILE)],
    )
    plsc.subcore_barrier()

    def load_meta(j, idx_s, idx_r, ef_b):
        base = (wid + j * NW) * C
        pltpu.sync_copy(send_hbm.at[pl.ds(base, C)], idx_s)
        pltpu.sync_copy(recv_hbm.at[pl.ds(base, C)], idx_r)
        pltpu.sync_copy(ef_hbm.at[pl.ds(base * D_EDGE, C * D_EDGE)], ef_b)

    def compute(prow, ef_b, idx_r):
        def group_body(q, _):
            v = ef_b[pl.ds(16 * q, 16)]
            for r in range(4):
                e = 4 * q + r
                f0 = v[4 * r + 0]
                f1 = v[4 * r + 1]
                f2 = v[4 * r + 2]
                f3 = v[4 * r + 3]
                for kg in range(8):
                    o = kg * 16
                    w = f0 * prow[e, pl.ds(o, 16)]
                    w = w + f1 * prow[e, pl.ds(128 + o, 16)]
                    w = w + f2 * prow[e, pl.ds(256 + o, 16)]
                    w = w + f3 * prow[e, pl.ds(384 + o, 16)]
                    outb[e, pl.ds(o, 16)] = w
            return 0

        lax.fori_loop(0, C // 4, group_body, 0)
        pltpu.sync_copy(outb, acc.at[idx_r], add=True)

    load_meta(0, idx_s0, idx_r0, ef_b0)
    pltpu.async_copy(p_hbm.at[idx_s0], prow0, sem0)

    def pair_body(i2, _):
        j0 = 2 * i2
        load_meta(j0 + 1, idx_s1, idx_r1, ef_b1)
        pltpu.async_copy(p_hbm.at[idx_s1], prow1, sem1)
        pltpu.make_async_copy(p_hbm.at[idx_s0], prow0, sem0).wait()
        compute(prow0, ef_b0, idx_r0)

        @pl.when(i2 + 1 < NPC // 2)
        def _():
            load_meta(j0 + 2, idx_s0, idx_r0, ef_b0)
            pltpu.async_copy(p_hbm.at[idx_s0], prow0, sem0)

        pltpu.make_async_copy(p_hbm.at[idx_s1], prow1, sem1).wait()
        compute(prow1, ef_b1, idx_r1)
        return 0

    lax.fori_loop(0, NPC // 2, pair_body, 0)

    plsc.subcore_barrier()
    pltpu.sync_copy(
        acc.at[pl.ds(sid * ROWS_PER_TILE, ROWS_PER_TILE)],
        out_hbm.at[cid, pl.ds(sid * ROWS_PER_TILE, ROWS_PER_TILE)],
    )


@functools.lru_cache(maxsize=1)
def _sc_scatter():
    return pl.kernel(
        _sc_body,
        out_type=jax.ShapeDtypeStruct((NC, NPAD, D_MID), jnp.float32),
        mesh=plsc.VectorSubcoreMesh(
            core_axis_name="c", subcore_axis_name="s",
            num_cores=NC, num_subcores=NS,
        ),
        scratch_types=[
            pltpu.VMEM((C,), jnp.int32),
            pltpu.VMEM((C,), jnp.int32),
            pltpu.VMEM((C,), jnp.int32),
            pltpu.VMEM((C,), jnp.int32),
            pltpu.VMEM((C * D_EDGE,), jnp.float32),
            pltpu.VMEM((C * D_EDGE,), jnp.float32),
            pltpu.VMEM((C, D_EDGE * D_MID), jnp.float32),
            pltpu.VMEM((C, D_EDGE * D_MID), jnp.float32),
            pltpu.VMEM((C, D_MID), jnp.float32),
            pltpu.VMEM_SHARED((NPAD, D_MID), jnp.float32),
            pltpu.SemaphoreType.DMA,
            pltpu.SemaphoreType.DMA,
        ],
    )


def _final_body(p0_ref, p1_ref, attr_ref, wl_ref, wsk_ref, o_ref):
    x = (
        jnp.dot(
            p0_ref[...] + p1_ref[...],
            wl_ref[...],
            preferred_element_type=jnp.float32,
        )
        * _INV_LIN
    )
    sk = jnp.zeros_like(x)
    for j in range(D_ATTR):
        sk = sk + jnp.dot(
            x * attr_ref[:, j : j + 1],
            wsk_ref[j],
            preferred_element_type=jnp.float32,
        )
    o_ref[...] = x + sk * _INV_SKIP


def _finalize(p0, p1, node_attrs, w_lin, wsk):
    bn = 400
    return pl.pallas_call(
        _final_body,
        grid=(N // bn,),
        in_specs=[
            pl.BlockSpec((bn, D_MID), lambda i: (i, 0)),
            pl.BlockSpec((bn, D_MID), lambda i: (i, 0)),
            pl.BlockSpec((bn, D_ATTR), lambda i: (i, 0)),
            pl.BlockSpec((D_MID, D_OUT), lambda i: (0, 0)),
            pl.BlockSpec((D_ATTR, D_MID, D_OUT), lambda i: (0, 0, 0)),
        ],
        out_specs=pl.BlockSpec((bn, D_OUT), lambda i: (i, 0)),
        out_shape=jax.ShapeDtypeStruct((N, D_OUT), jnp.float32),
    )(p0, p1, node_attrs, w_lin, wsk)


def kernel(node_feats, node_attrs, edge_feats, edge_index, W_conv, W_lin, W_skip):
    p = _conv_proj(node_feats, W_conv.reshape(D_NODE, D_EDGE * D_MID))
    zeros = jnp.zeros((NPAD, D_MID), jnp.float32)
    epad = EPAD - E
    send = jnp.concatenate([edge_index[0], jnp.zeros((epad,), jnp.int32)])
    recv = jnp.concatenate([edge_index[1], jnp.zeros((epad,), jnp.int32)])
    ef_flat = jnp.concatenate(
        [edge_feats.reshape(-1), jnp.zeros((epad * D_EDGE,), jnp.float32)]
    )
    partial = _sc_scatter()(p, send, recv, ef_flat, zeros)
    return _finalize(
        partial[0], partial[1], node_attrs, W_lin, W_skip.transpose(1, 0, 2)
    )
